# baseline (speedup 1.0000x reference)
import functools
import math

import jax
import jax.numpy as jnp
from jax import lax
from jax.experimental import pallas as pl

H_IMG, W_IMG = 800.0, 1216.0
PRE = 2000
NPAD = 2048
POST = 1000
TH = 0.7
MIN_SZ = 1.0
CLIP = math.log(1000.0 / 16.0)
BLK = 256
NBLK = NPAD // BLK
OPAD = 1024


def _decode(anc4, dl4):
    x1, y1, x2, y2 = anc4
    dx, dy, dw, dh = dl4
    w = x2 - x1
    h = y2 - y1
    cx = x1 + 0.5 * w
    cy = y1 + 0.5 * h
    dwc = jnp.minimum(dw, CLIP)
    dhc = jnp.minimum(dh, CLIP)
    pcx = dx * w + cx
    pcy = dy * h + cy
    pw = jnp.exp(dwc) * w
    ph = jnp.exp(dhc) * h
    px1 = jnp.clip(pcx - 0.5 * pw, 0.0, W_IMG)
    py1 = jnp.clip(pcy - 0.5 * ph, 0.0, H_IMG)
    px2 = jnp.clip(pcx + 0.5 * pw, 0.0, W_IMG)
    py2 = jnp.clip(pcy + 0.5 * ph, 0.0, H_IMG)
    return px1, py1, px2, py2


def _nms_body(anc_c_ref, del_c_ref, anc_r_ref, del_r_ref, out_ref):
    ar = tuple(anc_r_ref[0, k:k + 1, :] for k in range(4))
    dr = tuple(del_r_ref[0, k:k + 1, :] for k in range(4))
    rx1, ry1, rx2, ry2 = _decode(ar, dr)
    area_r = (rx2 - rx1) * (ry2 - ry1)
    cx1, cy1, cx2, cy2 = _decode(
        tuple(anc_c_ref[0][:, k:k + 1] for k in range(4)),
        tuple(del_c_ref[0][:, k:k + 1] for k in range(4)))

    col = lax.broadcasted_iota(jnp.int32, (1, NPAD), 1)
    real = col < PRE
    valid = (rx2 - rx1 >= MIN_SZ) & (ry2 - ry1 >= MIN_SZ) & real
    keep0 = valid.astype(jnp.float32)

    def block_step(r0, keep):
        bx1 = cx1[r0:r0 + BLK]
        by1 = cy1[r0:r0 + BLK]
        bx2 = cx2[r0:r0 + BLK]
        by2 = cy2[r0:r0 + BLK]
        area_b = (bx2 - bx1) * (by2 - by1)
        iw = jnp.maximum(jnp.minimum(bx2, rx2) - jnp.maximum(bx1, rx1), 0.0)
        ih = jnp.maximum(jnp.minimum(by2, ry2) - jnp.maximum(by1, ry1), 0.0)
        inter = iw * ih
        iou = inter / (area_b + area_r - inter + 1e-9)
        rowid = r0 + lax.broadcasted_iota(jnp.int32, (BLK, 1), 0)
        suppf = ((iou > TH) & (col > rowid)).astype(jnp.float32)
        sblk = suppf[:, r0:r0 + BLK]
        kb0 = keep[:, r0:r0 + BLK]

        def fix_cond(c):
            return c[1]

        def fix_body(c):
            kb, _ = c
            cnt = jnp.dot(kb, sblk, preferred_element_type=jnp.float32)
            kb2 = kb0 * (cnt < 0.5).astype(jnp.float32)
            return kb2, jnp.sum(jnp.abs(kb2 - kb)) > 0.0

        kb, _ = lax.while_loop(fix_cond, fix_body, (kb0, jnp.asarray(True)))
        cnt_all = jnp.dot(kb, suppf, preferred_element_type=jnp.float32)
        return keep * (cnt_all < 0.5).astype(jnp.float32)

    keep = keep0
    for b in range(NBLK):
        keep = block_step(b * BLK, keep)

    realf = real.astype(jnp.float32)
    sup = realf - keep
    kcount = jnp.sum(keep)
    rowc = lax.broadcasted_iota(jnp.int32, (NPAD, 1), 0)
    tri = (rowc < col).astype(jnp.float32)
    cntk = jnp.dot(keep, tri, preferred_element_type=jnp.float32)
    cnts = jnp.dot(sup, tri, preferred_element_type=jnp.float32)
    pos = jnp.where(keep > 0.5, cntk, kcount + cnts)
    pos = jnp.where(real, pos, 2.0 * NPAD).astype(jnp.int32)
    orow = lax.broadcasted_iota(jnp.int32, (OPAD, 1), 0)
    onehot = (orow == pos).astype(jnp.float32)
    boxc = jnp.concatenate([cx1, cy1, cx2, cy2], axis=1)
    out_ref[0] = jnp.dot(onehot, boxc,
                         precision=jax.lax.Precision.HIGHEST,
                         preferred_element_type=jnp.float32)


@functools.partial(jax.jit)
def kernel(anchors, objectness, bbox_deltas):
    B = objectness.shape[0]
    top_vals, top_idx = lax.top_k(objectness, PRE)
    anc = jnp.take(anchors, top_idx, axis=0)
    dl = jnp.take_along_axis(bbox_deltas, top_idx[..., None], axis=1)
    pad = NPAD - PRE
    anc_c = jnp.pad(anc, ((0, 0), (0, pad), (0, 0)))
    del_c = jnp.pad(dl, ((0, 0), (0, pad), (0, 0)))
    anc_r = anc_c.transpose(0, 2, 1)
    del_r = del_c.transpose(0, 2, 1)

    out = pl.pallas_call(
        _nms_body,
        grid=(B,),
        in_specs=[
            pl.BlockSpec((1, NPAD, 4), lambda b: (b, 0, 0)),
            pl.BlockSpec((1, NPAD, 4), lambda b: (b, 0, 0)),
            pl.BlockSpec((1, 4, NPAD), lambda b: (b, 0, 0)),
            pl.BlockSpec((1, 4, NPAD), lambda b: (b, 0, 0)),
        ],
        out_specs=pl.BlockSpec((1, OPAD, 4), lambda b: (b, 0, 0)),
        out_shape=jax.ShapeDtypeStruct((B, OPAD, 4), jnp.float32),
    )(anc_c, del_c, anc_r, del_r)

    return out[:, :POST, :]


# speedup vs baseline: 1.0032x; 1.0032x over previous
import functools
import math

import jax
import jax.numpy as jnp
from jax import lax
from jax.experimental import pallas as pl
from jax.experimental.pallas import tpu as pltpu

H_IMG, W_IMG = 800.0, 1216.0
PRE = 2000
NPAD = 2048
POST = 1000
TH = 0.7
MIN_SZ = 1.0
CLIP = math.log(1000.0 / 16.0)
BLK = 256
NBLK = NPAD // BLK
OPAD = 1024


def _decode(anc4, dl4):
    x1, y1, x2, y2 = anc4
    dx, dy, dw, dh = dl4
    w = x2 - x1
    h = y2 - y1
    cx = x1 + 0.5 * w
    cy = y1 + 0.5 * h
    dwc = jnp.minimum(dw, CLIP)
    dhc = jnp.minimum(dh, CLIP)
    pcx = dx * w + cx
    pcy = dy * h + cy
    pw = jnp.exp(dwc) * w
    ph = jnp.exp(dhc) * h
    px1 = jnp.clip(pcx - 0.5 * pw, 0.0, W_IMG)
    py1 = jnp.clip(pcy - 0.5 * ph, 0.0, H_IMG)
    px2 = jnp.clip(pcx + 0.5 * pw, 0.0, W_IMG)
    py2 = jnp.clip(pcy + 0.5 * ph, 0.0, H_IMG)
    return px1, py1, px2, py2


def _nms_body(anc_c_ref, del_c_ref, anc_r_ref, del_r_ref, out_ref):
    ar = tuple(anc_r_ref[0, k:k + 1, :] for k in range(4))
    dr = tuple(del_r_ref[0, k:k + 1, :] for k in range(4))
    rx1, ry1, rx2, ry2 = _decode(ar, dr)
    area_r = (rx2 - rx1) * (ry2 - ry1)
    cx1, cy1, cx2, cy2 = _decode(
        tuple(anc_c_ref[0][:, k:k + 1] for k in range(4)),
        tuple(del_c_ref[0][:, k:k + 1] for k in range(4)))

    col = lax.broadcasted_iota(jnp.int32, (1, NPAD), 1)
    real = col < PRE
    valid = (rx2 - rx1 >= MIN_SZ) & (ry2 - ry1 >= MIN_SZ) & real
    keep0 = valid.astype(jnp.float32)

    def block_step(r0, keep):
        bx1 = cx1[r0:r0 + BLK]
        by1 = cy1[r0:r0 + BLK]
        bx2 = cx2[r0:r0 + BLK]
        by2 = cy2[r0:r0 + BLK]
        area_b = (bx2 - bx1) * (by2 - by1)
        iw = jnp.maximum(jnp.minimum(bx2, rx2) - jnp.maximum(bx1, rx1), 0.0)
        ih = jnp.maximum(jnp.minimum(by2, ry2) - jnp.maximum(by1, ry1), 0.0)
        inter = iw * ih
        iou = inter / (area_b + area_r - inter + 1e-9)
        rowid = r0 + lax.broadcasted_iota(jnp.int32, (BLK, 1), 0)
        suppf = ((iou > TH) & (col > rowid)).astype(jnp.float32)
        sblk = suppf[:, r0:r0 + BLK]
        kb0 = keep[:, r0:r0 + BLK]

        def fix_cond(c):
            return c[1]

        def fix_body(c):
            kb, _ = c
            cnt = jnp.dot(kb, sblk, preferred_element_type=jnp.float32)
            kb2 = kb0 * (cnt < 0.5).astype(jnp.float32)
            return kb2, jnp.sum(jnp.abs(kb2 - kb)) > 0.0

        kb, _ = lax.while_loop(fix_cond, fix_body, (kb0, jnp.asarray(True)))
        cnt_all = jnp.dot(kb, suppf, preferred_element_type=jnp.float32)
        return keep * (cnt_all < 0.5).astype(jnp.float32)

    keep = keep0
    for b in range(NBLK):
        keep = block_step(b * BLK, keep)

    realf = real.astype(jnp.float32)
    sup = realf - keep
    kcount = jnp.sum(keep)

    def excl_prefix(x):
        s = x
        d = 1
        while d < NPAD:
            s = s + jnp.where(col >= d, pltpu.roll(s, d, axis=1), 0.0)
            d *= 2
        return s - x

    cntk = excl_prefix(keep)
    cnts = excl_prefix(sup)
    pos = jnp.where(keep > 0.5, cntk, kcount + cnts)
    pos = jnp.where(real, pos, 2.0 * NPAD).astype(jnp.int32)
    orow = lax.broadcasted_iota(jnp.int32, (OPAD, 1), 0)
    onehot = (orow == pos).astype(jnp.float32)
    boxc = jnp.concatenate([cx1, cy1, cx2, cy2], axis=1)
    out_ref[0] = jnp.dot(onehot, boxc,
                         precision=jax.lax.Precision.HIGHEST,
                         preferred_element_type=jnp.float32)


@functools.partial(jax.jit)
def kernel(anchors, objectness, bbox_deltas):
    B = objectness.shape[0]
    top_vals, top_idx = lax.top_k(objectness, PRE)
    anc = jnp.take(anchors, top_idx, axis=0)
    dl = jnp.take_along_axis(bbox_deltas, top_idx[..., None], axis=1)
    pad = NPAD - PRE
    anc_c = jnp.pad(anc, ((0, 0), (0, pad), (0, 0)))
    del_c = jnp.pad(dl, ((0, 0), (0, pad), (0, 0)))
    anc_r = anc_c.transpose(0, 2, 1)
    del_r = del_c.transpose(0, 2, 1)

    out = pl.pallas_call(
        _nms_body,
        grid=(B,),
        in_specs=[
            pl.BlockSpec((1, NPAD, 4), lambda b: (b, 0, 0)),
            pl.BlockSpec((1, NPAD, 4), lambda b: (b, 0, 0)),
            pl.BlockSpec((1, 4, NPAD), lambda b: (b, 0, 0)),
            pl.BlockSpec((1, 4, NPAD), lambda b: (b, 0, 0)),
        ],
        out_specs=pl.BlockSpec((1, OPAD, 4), lambda b: (b, 0, 0)),
        out_shape=jax.ShapeDtypeStruct((B, OPAD, 4), jnp.float32),
    )(anc_c, del_c, anc_r, del_r)

    return out[:, :POST, :]


# speedup vs baseline: 1.2289x; 1.2250x over previous
import functools
import math

import jax
import jax.numpy as jnp
from jax import lax
from jax.experimental import pallas as pl
from jax.experimental.pallas import tpu as pltpu

H_IMG, W_IMG = 800.0, 1216.0
A_TOT = 20000
NCOLS = 20480
PRE = 2000
NPAD = 2048
POST = 1000
TH = 0.7
MIN_SZ = 1.0
CLIP = math.log(1000.0 / 16.0)
BLK = 256
NBLK = NPAD // BLK
OPAD = 1024
MININT = -2**31


def _select_body(obj_ref, sc_ref, idx_ref):
    s = obj_ref[...]
    nb = s.shape[0]
    w = lax.bitcast_convert_type(s, jnp.int32)
    m = jnp.where(w >= 0, w, w ^ 0x7FFFFFFF)
    u = m ^ MININT
    col = lax.broadcasted_iota(jnp.int32, (nb, NCOLS), 1)

    def radix_step(t, carry):
        active, prefix, k = carry
        bit = lax.shift_left(jnp.int32(1), 31 - t)
        bitset = (u & bit) != 0
        cnt1 = jnp.sum(active * bitset.astype(jnp.int32), axis=1,
                       keepdims=True)
        ge = cnt1 >= k
        active = active * jnp.where(bitset == ge, 1, 0)
        prefix = prefix | jnp.where(ge, bit, 0)
        k = jnp.where(ge, k, k - cnt1)
        return active, prefix, k

    init = (jnp.ones((nb, NCOLS), dtype=jnp.int32),
            jnp.zeros((nb, 1), dtype=jnp.int32),
            jnp.full((nb, 1), PRE, dtype=jnp.int32))
    _, prefix, k = lax.fori_loop(0, 32, radix_step, init)

    mt = prefix ^ MININT
    sel_gt = m > mt
    ties = m == mt

    def excl_prefix(x):
        t = x
        d = 1
        while d < NCOLS:
            t = t + jnp.where(col >= d, pltpu.roll(t, d, axis=1), 0.0)
            d *= 2
        return t - x

    tie_pref = excl_prefix(ties.astype(jnp.float32))
    sel = sel_gt | (ties & (tie_pref < k.astype(jnp.float32)))
    pos = excl_prefix(sel.astype(jnp.float32)).astype(jnp.int32)
    shift = col - pos

    s_c = s
    idx_c = col.astype(jnp.float32)
    sh_c = shift
    d = 1
    while d <= 1 << 14:
        sh_r = pltpu.roll(sh_c, NCOLS - d, axis=1)
        cond = (sh_r & d) != 0
        s_c = jnp.where(cond, pltpu.roll(s_c, NCOLS - d, axis=1), s_c)
        idx_c = jnp.where(cond, pltpu.roll(idx_c, NCOLS - d, axis=1), idx_c)
        sh_c = jnp.where(cond, sh_r - d, sh_c)
        d *= 2

    sc_ref[...] = s_c[:, :NPAD]
    idx_ref[...] = idx_c[:, :NPAD]


def _decode(anc4, dl4):
    x1, y1, x2, y2 = anc4
    dx, dy, dw, dh = dl4
    w = x2 - x1
    h = y2 - y1
    cx = x1 + 0.5 * w
    cy = y1 + 0.5 * h
    dwc = jnp.minimum(dw, CLIP)
    dhc = jnp.minimum(dh, CLIP)
    pcx = dx * w + cx
    pcy = dy * h + cy
    pw = jnp.exp(dwc) * w
    ph = jnp.exp(dhc) * h
    px1 = jnp.clip(pcx - 0.5 * pw, 0.0, W_IMG)
    py1 = jnp.clip(pcy - 0.5 * ph, 0.0, H_IMG)
    px2 = jnp.clip(pcx + 0.5 * pw, 0.0, W_IMG)
    py2 = jnp.clip(pcy + 0.5 * ph, 0.0, H_IMG)
    return px1, py1, px2, py2


def _nms_body(anc_c_ref, del_c_ref, anc_r_ref, del_r_ref, sc_c_ref, sc_r_ref,
              ix_c_ref, ix_r_ref, out_ref, supp_ref, cmp_ref):
    ar = tuple(anc_r_ref[0, k:k + 1, :] for k in range(4))
    dr = tuple(del_r_ref[0, k:k + 1, :] for k in range(4))
    rx1, ry1, rx2, ry2 = _decode(ar, dr)
    area_r = (rx2 - rx1) * (ry2 - ry1)
    cx1, cy1, cx2, cy2 = _decode(
        tuple(anc_c_ref[0][:, k:k + 1] for k in range(4)),
        tuple(del_c_ref[0][:, k:k + 1] for k in range(4)))
    s_r = sc_r_ref[0, 0:1, :]
    s_col = sc_c_ref[0]
    i_r = ix_r_ref[0, 0:1, :]
    i_col = ix_c_ref[0]

    col = lax.broadcasted_iota(jnp.int32, (1, NPAD), 1)
    real = col < PRE
    valid = (rx2 - rx1 >= MIN_SZ) & (ry2 - ry1 >= MIN_SZ) & real
    keep0 = valid.astype(jnp.float32)

    for b in range(NBLK):
        r0 = b * BLK
        bx1 = cx1[r0:r0 + BLK]
        by1 = cy1[r0:r0 + BLK]
        bx2 = cx2[r0:r0 + BLK]
        by2 = cy2[r0:r0 + BLK]
        s_b = s_col[r0:r0 + BLK]
        i_b = i_col[r0:r0 + BLK]
        area_b = (bx2 - bx1) * (by2 - by1)
        iw = jnp.maximum(jnp.minimum(bx2, rx2) - jnp.maximum(bx1, rx1), 0.0)
        ih = jnp.maximum(jnp.minimum(by2, ry2) - jnp.maximum(by1, ry1), 0.0)
        inter = iw * ih
        iou = inter / (area_b + area_r - inter + 1e-9)
        cmp_b = (s_b > s_r) | ((s_b == s_r) & (i_b < i_r))
        cmp_ref[r0:r0 + BLK, :] = cmp_b.astype(jnp.float32)
        supp_ref[r0:r0 + BLK, :] = ((iou > TH) & cmp_b).astype(jnp.float32)

    supp = supp_ref[...]
    cmp = cmp_ref[...]

    def fix_cond(c):
        return c[1]

    def fix_body(c):
        kb, _ = c
        cnt = jnp.dot(kb, supp, preferred_element_type=jnp.float32)
        kb2 = keep0 * (cnt < 0.5).astype(jnp.float32)
        return kb2, jnp.sum(jnp.abs(kb2 - kb)) > 0.0

    keep, _ = lax.while_loop(fix_cond, fix_body, (keep0, jnp.asarray(True)))

    realf = real.astype(jnp.float32)
    sup = realf - keep
    kcount = jnp.sum(keep)
    cntk = jnp.dot(keep, cmp, preferred_element_type=jnp.float32)
    cnts = jnp.dot(sup, cmp, preferred_element_type=jnp.float32)
    pos = jnp.where(keep > 0.5, cntk, kcount + cnts)
    pos = jnp.where(real, pos, 2.0 * NPAD).astype(jnp.int32)
    orow = lax.broadcasted_iota(jnp.int32, (OPAD, 1), 0)
    onehot = (orow == pos).astype(jnp.float32)
    boxc = jnp.concatenate([cx1, cy1, cx2, cy2], axis=1)
    out_ref[0] = jnp.dot(onehot, boxc,
                         precision=jax.lax.Precision.HIGHEST,
                         preferred_element_type=jnp.float32)


@functools.partial(jax.jit)
def kernel(anchors, objectness, bbox_deltas):
    B, A = objectness.shape
    obj_p = jnp.pad(objectness, ((0, 0), (0, NCOLS - A)),
                    constant_values=-jnp.inf)

    sc2, idxf = pl.pallas_call(
        _select_body,
        in_specs=[pl.BlockSpec((B, NCOLS), lambda: (0, 0))],
        out_specs=[
            pl.BlockSpec((B, NPAD), lambda: (0, 0)),
            pl.BlockSpec((B, NPAD), lambda: (0, 0)),
        ],
        out_shape=[
            jax.ShapeDtypeStruct((B, NPAD), jnp.float32),
            jax.ShapeDtypeStruct((B, NPAD), jnp.float32),
        ],
    )(obj_p)

    idxf = jnp.clip(idxf, 0.0, float(A - 1))
    idx = idxf.astype(jnp.int32)
    anc_c = jnp.take(anchors, idx, axis=0)
    del_c = jnp.take_along_axis(bbox_deltas, idx[..., None], axis=1)
    anc_r = anc_c.transpose(0, 2, 1)
    del_r = del_c.transpose(0, 2, 1)

    out = pl.pallas_call(
        _nms_body,
        grid=(B,),
        in_specs=[
            pl.BlockSpec((1, NPAD, 4), lambda b: (b, 0, 0)),
            pl.BlockSpec((1, NPAD, 4), lambda b: (b, 0, 0)),
            pl.BlockSpec((1, 4, NPAD), lambda b: (b, 0, 0)),
            pl.BlockSpec((1, 4, NPAD), lambda b: (b, 0, 0)),
            pl.BlockSpec((1, NPAD, 1), lambda b: (b, 0, 0)),
            pl.BlockSpec((1, 1, NPAD), lambda b: (b, 0, 0)),
            pl.BlockSpec((1, NPAD, 1), lambda b: (b, 0, 0)),
            pl.BlockSpec((1, 1, NPAD), lambda b: (b, 0, 0)),
        ],
        out_specs=pl.BlockSpec((1, OPAD, 4), lambda b: (b, 0, 0)),
        out_shape=jax.ShapeDtypeStruct((B, OPAD, 4), jnp.float32),
        scratch_shapes=[
            pltpu.VMEM((NPAD, NPAD), jnp.float32),
            pltpu.VMEM((NPAD, NPAD), jnp.float32),
        ],
    )(anc_c, del_c, anc_r, del_r,
      sc2[..., None], sc2[:, None, :], idxf[..., None], idxf[:, None, :])

    return out[:, :POST, :]
